# Initial kernel scaffold; baseline (speedup 1.0000x reference)
import jax
import jax.numpy as jnp
from jax.experimental import pallas as pl


def kernel(x, node_w1, node_b1, node_w2, node_b2, edge_w1, edge_b1, edge_w2, edge_b2, v2e_w1, v2e_b1, v2e_w2, v2e_b2, e2v_w1, e2v_b1, e2v_w2, e2v_b2, head_w1, head_b1, head_w2, head_b2, scale):
    raise NotImplementedError("write your pallas kernel here")


# speedup vs baseline: 3.4937x; 3.4937x over previous
import math

import jax
import jax.numpy as jnp
import numpy as np
from jax.experimental import pallas as pl
from jax.experimental.pallas import tpu as pltpu

_N = 96
_P = _N * _N
_L = 2
_HIGH = jax.lax.Precision.HIGHEST


def _gelu(t):
    return t * 0.5 * (1.0 + jax.lax.erf(t * np.float32(1.0 / math.sqrt(2.0))))


def _dot(a, b, precision=_HIGH):
    return jax.lax.dot_general(a, b, (((1,), (0,)), ((), ())),
                               precision=precision,
                               preferred_element_type=jnp.float32)


def _mlp(w1_ref, b1_ref, w2_ref, b2_ref, act):
    h = _gelu(_dot(w1_ref[...], act, None) + b1_ref[...])
    return _dot(w2_ref[...], h, None) + b2_ref[...]


def _body(xT_ref, nw1t_ref, nb1_ref, nw2t_ref, nb2_ref,
          ew1t_ref, eb1_ref, ew2t_ref, eb2_ref,
          v2ew1t_ref, v2eb1_ref, v2ew2t_ref, v2eb2_ref,
          e2vw1t_ref, e2vb1_ref, e2vw2t_ref, e2vb2_ref,
          hw1t_ref, hb1_ref, hw2t_ref, hb2_ref, scale_ref,
          T_ref, Mdr_ref, A_ref, out_ref):
    X = xT_ref[0]

    h_v = _mlp(nw1t_ref, nb1_ref, nw2t_ref, nb2_ref, X)

    dr = _dot(X, Mdr_ref[...])
    r2 = jnp.sum(dr * dr, axis=0, keepdims=True)
    rr = jnp.sqrt(r2 + 1e-12)
    e_feat = jnp.concatenate([dr, rr, r2], axis=0)
    h_e = _mlp(ew1t_ref, eb1_ref, ew2t_ref, eb2_ref, e_feat)

    Tm = T_ref[...]
    inv = jnp.float32(1.0 / (_N - 1))
    for l in range(_L):
        h_v_src = _dot(h_v, Tm)
        cat = jnp.concatenate([h_v_src, h_e], axis=0)
        h_e = _mlp(v2ew1t_ref[l], v2eb1_ref[l], v2ew2t_ref[l], v2eb2_ref[l], cat)
        msgs = _mlp(e2vw1t_ref[l], e2vb1_ref[l], e2vw2t_ref[l], e2vb2_ref[l], h_e)
        agg = _dot(msgs, A_ref[...])
        h_v = h_v + agg * inv

    t = jnp.tanh(_dot(hw1t_ref[...], h_v, None) + hb1_ref[...])
    dx = _dot(hw2t_ref[...], t, None) + hb2_ref[...]
    sp = jnp.log1p(jnp.exp(scale_ref[...]))
    dx = dx * sp
    dx = dx - jnp.mean(dx, axis=1, keepdims=True)
    out_ref[0] = dx


def kernel(x, node_w1, node_b1, node_w2, node_b2, edge_w1, edge_b1, edge_w2,
           edge_b2, v2e_w1, v2e_b1, v2e_w2, v2e_b2, e2v_w1, e2v_b1, e2v_w2,
           e2v_b2, head_w1, head_b1, head_w2, head_b2, scale):
    B, N, D = x.shape
    omega = 1.0
    xT = jnp.transpose(x, (0, 2, 1)) * np.float32(math.sqrt(omega))

    eye = np.eye(_N, dtype=np.float32)
    T = np.tile(eye, (1, _N))
    Rep = np.kron(eye, np.ones((1, _N), dtype=np.float32))
    Mdr = Rep - T
    A = np.kron(eye, np.ones((_N, 1), dtype=np.float32))
    A[np.arange(_N) * _N + np.arange(_N), np.arange(_N)] = 0.0

    f32 = jnp.float32
    args = (
        xT,
        node_w1[:D].T.astype(f32), node_b1[:, None],
        node_w2.T, node_b2[:, None],
        edge_w1.T, edge_b1[:, None],
        edge_w2.T, edge_b2[:, None],
        jnp.transpose(v2e_w1, (0, 2, 1)), v2e_b1[:, :, None],
        jnp.transpose(v2e_w2, (0, 2, 1)), v2e_b2[:, :, None],
        jnp.transpose(e2v_w1, (0, 2, 1)), e2v_b1[:, :, None],
        jnp.transpose(e2v_w2, (0, 2, 1)), e2v_b2[:, :, None],
        head_w1.T, head_b1[:, None],
        head_w2.T, head_b2[:, None],
        jnp.reshape(scale, (1, 1)),
        jnp.asarray(T), jnp.asarray(Mdr), jnp.asarray(A),
    )

    def full(a):
        return pl.BlockSpec(a.shape, lambda b, _nd=a.ndim: (0,) * _nd)

    in_specs = [pl.BlockSpec((1, D, N), lambda b: (b, 0, 0))]
    in_specs += [full(a) for a in args[1:]]

    out = pl.pallas_call(
        _body,
        grid=(B,),
        in_specs=in_specs,
        out_specs=pl.BlockSpec((1, D, N), lambda b: (b, 0, 0)),
        out_shape=jax.ShapeDtypeStruct((B, D, N), jnp.float32),
        compiler_params=pltpu.CompilerParams(
            dimension_semantics=("parallel",),
        ),
    )(*args)
    return jnp.transpose(out, (0, 2, 1))


# speedup vs baseline: 7.4812x; 2.1414x over previous
import math

import jax
import jax.numpy as jnp
import numpy as np
from jax.experimental import pallas as pl
from jax.experimental.pallas import tpu as pltpu

_N = 96
_P = _N * _N
_L = 2
_HIGH = jax.lax.Precision.HIGHEST


def _gelu(t):
    return t * 0.5 * (1.0 + jax.lax.erf(t * np.float32(1.0 / math.sqrt(2.0))))


def _dot(a, b, precision=_HIGH):
    return jax.lax.dot_general(a, b, (((1,), (0,)), ((), ())),
                               precision=precision,
                               preferred_element_type=jnp.float32)


def _mlp(w1_ref, b1_ref, w2_ref, b2_ref, act):
    h = _gelu(_dot(w1_ref[...], act, None) + b1_ref[...])
    return _dot(w2_ref[...], h, None) + b2_ref[...]


def _body(xT_ref, nw1t_ref, nb1_ref, nw2t_ref, nb2_ref,
          ew1t_ref, eb1_ref, ew2t_ref, eb2_ref,
          v2ew1t_ref, v2eb1_ref, v2ew2t_ref, v2eb2_ref,
          e2vw1t_ref, e2vb1_ref, e2vw2t_ref, e2vb2_ref,
          hw1t_ref, hb1_ref, hw2t_ref, hb2_ref, scale_ref,
          T_ref, Mdr_ref, A_ref, out_ref):
    X = xT_ref[0]

    h_v = _mlp(nw1t_ref, nb1_ref, nw2t_ref, nb2_ref, X)

    dr = _dot(X, Mdr_ref[...])
    r2 = jnp.sum(dr * dr, axis=0, keepdims=True)
    rr = jnp.sqrt(r2 + 1e-12)
    e_feat = jnp.concatenate([dr, rr, r2], axis=0)
    h_e = _mlp(ew1t_ref, eb1_ref, ew2t_ref, eb2_ref, e_feat)

    Tm = T_ref[...]
    inv = jnp.float32(1.0 / (_N - 1))
    for l in range(_L):
        h_v_src = _dot(h_v, Tm, None)
        cat = jnp.concatenate([h_v_src, h_e], axis=0)
        h_e = _mlp(v2ew1t_ref[l], v2eb1_ref[l], v2ew2t_ref[l], v2eb2_ref[l], cat)
        msgs = _mlp(e2vw1t_ref[l], e2vb1_ref[l], e2vw2t_ref[l], e2vb2_ref[l], h_e)
        agg = _dot(msgs, A_ref[...], None)
        h_v = h_v + agg * inv

    t = jnp.tanh(_dot(hw1t_ref[...], h_v, None) + hb1_ref[...])
    dx = _dot(hw2t_ref[...], t, None) + hb2_ref[...]
    sp = jnp.log1p(jnp.exp(scale_ref[...]))
    dx = dx * sp
    dx = dx - jnp.mean(dx, axis=1, keepdims=True)
    out_ref[0] = dx


def kernel(x, node_w1, node_b1, node_w2, node_b2, edge_w1, edge_b1, edge_w2,
           edge_b2, v2e_w1, v2e_b1, v2e_w2, v2e_b2, e2v_w1, e2v_b1, e2v_w2,
           e2v_b2, head_w1, head_b1, head_w2, head_b2, scale):
    B, N, D = x.shape
    omega = 1.0
    xT = jnp.transpose(x, (0, 2, 1)) * np.float32(math.sqrt(omega))

    eye = np.eye(_N, dtype=np.float32)
    T = np.tile(eye, (1, _N))
    Rep = np.kron(eye, np.ones((1, _N), dtype=np.float32))
    Mdr = Rep - T
    A = np.kron(eye, np.ones((_N, 1), dtype=np.float32))
    A[np.arange(_N) * _N + np.arange(_N), np.arange(_N)] = 0.0

    f32 = jnp.float32
    args = (
        xT,
        node_w1[:D].T.astype(f32), node_b1[:, None],
        node_w2.T, node_b2[:, None],
        edge_w1.T, edge_b1[:, None],
        edge_w2.T, edge_b2[:, None],
        jnp.transpose(v2e_w1, (0, 2, 1)), v2e_b1[:, :, None],
        jnp.transpose(v2e_w2, (0, 2, 1)), v2e_b2[:, :, None],
        jnp.transpose(e2v_w1, (0, 2, 1)), e2v_b1[:, :, None],
        jnp.transpose(e2v_w2, (0, 2, 1)), e2v_b2[:, :, None],
        head_w1.T, head_b1[:, None],
        head_w2.T, head_b2[:, None],
        jnp.reshape(scale, (1, 1)),
        jnp.asarray(T), jnp.asarray(Mdr), jnp.asarray(A),
    )

    def full(a):
        return pl.BlockSpec(a.shape, lambda b, _nd=a.ndim: (0,) * _nd)

    in_specs = [pl.BlockSpec((1, D, N), lambda b: (b, 0, 0))]
    in_specs += [full(a) for a in args[1:]]

    out = pl.pallas_call(
        _body,
        grid=(B,),
        in_specs=in_specs,
        out_specs=pl.BlockSpec((1, D, N), lambda b: (b, 0, 0)),
        out_shape=jax.ShapeDtypeStruct((B, D, N), jnp.float32),
        compiler_params=pltpu.CompilerParams(
            dimension_semantics=("parallel",),
        ),
    )(*args)
    return jnp.transpose(out, (0, 2, 1))
